# baseline (speedup 1.0000x reference)
import functools

import jax
import jax.numpy as jnp
from jax import lax
from jax.experimental import pallas as pl
from jax.experimental.pallas import tpu as pltpu
from jax.experimental.pallas import tpu_sc as plsc

B = 128
K, H, W = 5, 25, 25
NQ = 4
C = NQ * K
TOT = B * K * H * W

_info = plsc.get_sparse_core_info()
NC, NS, L = _info.num_cores, _info.num_subcores, _info.num_lanes
NW = NC * NS
ITERS = -(-(TOT // NW) // L)
PERW = ITERS * L
LASTBASE = TOT - PERW
LASTLO = (NW - 1) * PERW - LASTBASE


def _cls_sc_kernel(label_hbm, pred_hbm, out_hbm, lab_v, pred_v, res_v):
    wid = lax.axis_index("s") * NC + lax.axis_index("c")
    base = jnp.minimum(wid * PERW, LASTBASE)
    lo = jnp.where(wid == NW - 1, LASTLO, 0)
    pltpu.sync_copy(label_hbm.at[pl.ds(base, PERW)], lab_v)
    pltpu.sync_copy(pred_hbm.at[pl.ds(2 * base, 2 * PERW)], pred_v)

    zero = jnp.zeros((L,), jnp.float32)

    def body(i, carry):
        sp, sn, cp = carry
        off = i * L + lax.iota(jnp.int32, L)
        lab = lab_v[pl.ds(i * L, L)]
        idx = jnp.minimum(2 * off + lab, 2 * PERW - 1)
        sel = plsc.load_gather(pred_v, [idx])
        valid = off >= lo
        pos = valid & (lab == 1)
        neg = valid & (lab == 0)
        sp = sp + jnp.where(pos, sel, 0.0)
        sn = sn + jnp.where(neg, sel, 0.0)
        cp = cp + jnp.where(pos, 1.0, 0.0)
        return sp, sn, cp

    sp, sn, cp = lax.fori_loop(0, ITERS, body, (zero, zero, zero))
    res_v[pl.ds(0, L)] = sp
    res_v[pl.ds(L, L)] = sn
    res_v[pl.ds(2 * L, L)] = cp
    pltpu.sync_copy(res_v, out_hbm.at[pl.ds(wid * 3 * L, 3 * L)])


_cls_sc = functools.partial(
    pl.kernel,
    _cls_sc_kernel,
    out_type=jax.ShapeDtypeStruct((NW * 3 * L,), jnp.float32),
    mesh=plsc.VectorSubcoreMesh(core_axis_name="c", subcore_axis_name="s"),
    compiler_params=pltpu.CompilerParams(needs_layout_passes=False),
    scratch_types=[
        pltpu.VMEM((PERW,), jnp.int32),
        pltpu.VMEM((2 * PERW,), jnp.float32),
        pltpu.VMEM((3 * L,), jnp.float32),
    ],
)


def _loc_kernel(ploc_ref, lloc_ref, w_ref, out_ref, plocx, llocx, sem):
    copies = []
    for c in range(C):
        q, kk = divmod(c, K)
        cp = pltpu.make_async_copy(ploc_ref.at[:, :, c, :], plocx.at[c], sem)
        cl = pltpu.make_async_copy(lloc_ref.at[kk, :, :, q, :], llocx.at[c],
                                   sem)
        cp.start()
        cl.start()
        copies.append(cp)
        copies.append(cl)
    for cp in copies:
        cp.wait()

    wv = w_ref[...]
    acc = jnp.zeros((H, W, B), jnp.float32)
    for c in range(C):
        acc = acc + jnp.abs(plocx[c] - llocx[c]) * wv[c % K]
    out_ref[0, 0] = jnp.sum(acc)


def kernel(label_cls, label_loc, label_loc_weight, rpn_pred_cls, rpn_pred_loc):
    label_f = label_cls.reshape(TOT)
    pred_f = rpn_pred_cls.reshape(2 * TOT)
    parts = _cls_sc()(label_f, pred_f)

    ploc = jnp.transpose(rpn_pred_loc, (2, 3, 1, 0))
    lloc = jnp.transpose(label_loc, (2, 3, 4, 1, 0))
    w = jnp.transpose(label_loc_weight, (1, 2, 3, 0))
    loc = pl.pallas_call(
        _loc_kernel,
        in_specs=[
            pl.BlockSpec(memory_space=pltpu.HBM),
            pl.BlockSpec(memory_space=pltpu.HBM),
            pl.BlockSpec((K, H, W, B), lambda: (0, 0, 0, 0)),
        ],
        out_specs=pl.BlockSpec(memory_space=pltpu.SMEM),
        out_shape=jax.ShapeDtypeStruct((1, 1), jnp.float32),
        scratch_shapes=[
            pltpu.VMEM((C, H, W, B), jnp.float32),
            pltpu.VMEM((C, H, W, B), jnp.float32),
            pltpu.SemaphoreType.DMA,
        ],
    )(ploc, lloc, w)[0, 0]

    sums = parts.reshape(NW, 3, L).sum(axis=(0, 2))
    cnt_pos = sums[2]
    cnt_neg = TOT - cnt_pos
    loss_pos = -sums[0] / jnp.maximum(cnt_pos, 1.0)
    loss_neg = -sums[1] / jnp.maximum(cnt_neg, 1.0)
    return 0.5 * loss_pos + 0.5 * loss_neg + 1.2 * (loc / B)


# speedup vs baseline: 7.4668x; 7.4668x over previous
import jax
import jax.numpy as jnp
from jax.experimental import pallas as pl
from jax.experimental.pallas import tpu as pltpu

B = 128
K, H, W = 5, 25, 25
NQ = 4
C = NQ * K


def _loss_kernel(label_ref, pred_ref, ploc_ref, lloc_ref, w_ref, out_ref,
                 plocx, llocx, sem):
    copies = []
    for c in range(C):
        q, kk = divmod(c, K)
        cp = pltpu.make_async_copy(ploc_ref.at[:, :, c, :], plocx.at[c], sem)
        cl = pltpu.make_async_copy(lloc_ref.at[kk, :, :, q, :], llocx.at[c],
                                   sem)
        cp.start()
        cl.start()
        copies.append(cp)
        copies.append(cl)

    label = label_ref[...]
    posf = (label == 1).astype(jnp.float32)
    negf = (label == 0).astype(jnp.float32)
    cnt_pos = jnp.sum(posf)
    cnt_neg = jnp.sum(negf)

    r_i = jax.lax.broadcasted_iota(jnp.int32, (2 * W, W), 0)
    w_i = jax.lax.broadcasted_iota(jnp.int32, (2 * W, W), 1)
    e1 = (r_i == 2 * w_i + 1).astype(jnp.bfloat16)
    e0 = (r_i == 2 * w_i).astype(jnp.bfloat16)
    dn = (((1,), (0,)), ((), ()))

    posb = posf.astype(jnp.bfloat16)
    negb = negf.astype(jnp.bfloat16)
    apos = jnp.zeros((W, B), jnp.float32)
    aneg = jnp.zeros((W, B), jnp.float32)
    for kk in range(K):
        for h in range(H):
            m1 = jax.lax.dot_general(e1, posb[kk, h], dn,
                                     preferred_element_type=jnp.float32)
            m0 = jax.lax.dot_general(e0, negb[kk, h], dn,
                                     preferred_element_type=jnp.float32)
            for g in (0, 1):
                eps, eta = divmod(2 * h + g, W)
                slab = pred_ref[2 * kk + eps, eta]
                apos = apos + slab * m1[g * W:(g + 1) * W]
                aneg = aneg + slab * m0[g * W:(g + 1) * W]
    sum_pos = jnp.sum(apos)
    sum_neg = jnp.sum(aneg)

    for cp in copies:
        cp.wait()

    wv = w_ref[...]
    acc = jnp.zeros((H, W, B), jnp.float32)
    for c in range(C):
        acc = acc + jnp.abs(plocx[c] - llocx[c]) * wv[c % K]
    loc = jnp.sum(acc)

    loss_pos = -sum_pos / jnp.maximum(cnt_pos, 1.0)
    loss_neg = -sum_neg / jnp.maximum(cnt_neg, 1.0)
    out_ref[0, 0] = 0.5 * loss_pos + 0.5 * loss_neg + 1.2 * (loc / B)


def kernel(label_cls, label_loc, label_loc_weight, rpn_pred_cls, rpn_pred_loc):
    label = jnp.transpose(label_cls, (1, 2, 3, 0))
    pred = jnp.transpose(rpn_pred_cls, (1, 2, 3, 0))
    ploc = jnp.transpose(rpn_pred_loc, (2, 3, 1, 0))
    lloc = jnp.transpose(label_loc, (2, 3, 4, 1, 0))
    w = jnp.transpose(label_loc_weight, (1, 2, 3, 0))

    out = pl.pallas_call(
        _loss_kernel,
        in_specs=[
            pl.BlockSpec((K, H, W, B), lambda: (0, 0, 0, 0)),
            pl.BlockSpec((2 * K, H, W, B), lambda: (0, 0, 0, 0)),
            pl.BlockSpec(memory_space=pltpu.HBM),
            pl.BlockSpec(memory_space=pltpu.HBM),
            pl.BlockSpec((K, H, W, B), lambda: (0, 0, 0, 0)),
        ],
        out_specs=pl.BlockSpec(memory_space=pltpu.SMEM),
        out_shape=jax.ShapeDtypeStruct((1, 1), jnp.float32),
        scratch_shapes=[
            pltpu.VMEM((C, H, W, B), jnp.float32),
            pltpu.VMEM((C, H, W, B), jnp.float32),
            pltpu.SemaphoreType.DMA,
        ],
    )(label, pred, ploc, lloc, w)
    return out[0, 0]


# speedup vs baseline: 8.3910x; 1.1238x over previous
import jax
import jax.numpy as jnp
from jax.experimental import pallas as pl
from jax.experimental.pallas import tpu as pltpu

B = 128
K, H, W = 5, 25, 25
NQ = 4
C = NQ * K


def _loss_kernel(label_ref, pred_ref, ploc_ref, lloc_ref, w_ref, out_ref,
                 plocx, llocx, sem):
    copies = []
    for c in range(C):
        q, kk = divmod(c, K)
        cp = pltpu.make_async_copy(ploc_ref.at[:, :, c, :], plocx.at[c], sem)
        cl = pltpu.make_async_copy(lloc_ref.at[kk, :, :, q, :], llocx.at[c],
                                   sem)
        cp.start()
        cl.start()
        copies.append(cp)
        copies.append(cl)

    label = label_ref[...]
    posf = (label == 1).astype(jnp.float32)
    negf = (label == 0).astype(jnp.float32)
    cnt_pos = jnp.sum(posf)
    cnt_neg = jnp.sum(negf)

    r_i = jax.lax.broadcasted_iota(jnp.int32, (2 * W, W), 0)
    w_i = jax.lax.broadcasted_iota(jnp.int32, (2 * W, W), 1)
    e1 = (r_i == 2 * w_i + 1).astype(jnp.bfloat16)
    e0 = (r_i == 2 * w_i).astype(jnp.bfloat16)
    dn = (((1,), (0,)), ((), ()))

    posb = posf.astype(jnp.bfloat16)
    negb = negf.astype(jnp.bfloat16)
    apos = jnp.zeros((W, B), jnp.float32)
    aneg = jnp.zeros((W, B), jnp.float32)
    for kk in range(K):
        for h in range(H):
            m1 = jax.lax.dot_general(e1, posb[kk, h], dn,
                                     preferred_element_type=jnp.float32)
            m0 = jax.lax.dot_general(e0, negb[kk, h], dn,
                                     preferred_element_type=jnp.float32)
            for g in (0, 1):
                eps, eta = divmod(2 * h + g, W)
                slab = pred_ref[2 * kk + eps, eta]
                apos = apos + slab * m1[g * W:(g + 1) * W]
                aneg = aneg + slab * m0[g * W:(g + 1) * W]
    sum_pos = jnp.sum(apos)
    sum_neg = jnp.sum(aneg)

    for cp in copies:
        cp.wait()

    wv = w_ref[...]
    loc = jnp.float32(0.0)
    for c in range(C):
        loc = loc + jnp.sum(jnp.abs(plocx[c] - llocx[c]) * wv[c % K])

    loss_pos = -sum_pos / jnp.maximum(cnt_pos, 1.0)
    loss_neg = -sum_neg / jnp.maximum(cnt_neg, 1.0)
    out_ref[0, 0] = 0.5 * loss_pos + 0.5 * loss_neg + 1.2 * (loc / B)


def kernel(label_cls, label_loc, label_loc_weight, rpn_pred_cls, rpn_pred_loc):
    label = jnp.transpose(label_cls, (1, 2, 3, 0))
    pred = jnp.transpose(rpn_pred_cls, (1, 2, 3, 0))
    ploc = jnp.transpose(rpn_pred_loc, (2, 3, 1, 0))
    lloc = jnp.transpose(label_loc, (2, 3, 4, 1, 0))
    w = jnp.transpose(label_loc_weight, (1, 2, 3, 0))

    out = pl.pallas_call(
        _loss_kernel,
        in_specs=[
            pl.BlockSpec((K, H, W, B), lambda: (0, 0, 0, 0)),
            pl.BlockSpec((2 * K, H, W, B), lambda: (0, 0, 0, 0)),
            pl.BlockSpec(memory_space=pltpu.HBM),
            pl.BlockSpec(memory_space=pltpu.HBM),
            pl.BlockSpec((K, H, W, B), lambda: (0, 0, 0, 0)),
        ],
        out_specs=pl.BlockSpec(memory_space=pltpu.SMEM),
        out_shape=jax.ShapeDtypeStruct((1, 1), jnp.float32),
        scratch_shapes=[
            pltpu.VMEM((C, H, W, B), jnp.float32),
            pltpu.VMEM((C, H, W, B), jnp.float32),
            pltpu.SemaphoreType.DMA,
        ],
    )(label, pred, ploc, lloc, w)
    return out[0, 0]
